# baseline (speedup 1.0000x reference)
import functools

import jax
import jax.numpy as jnp
from jax import lax
from jax.experimental import pallas as pl
from jax.experimental.pallas import tpu as pltpu
from jax.experimental.pallas import tpu_sc as plsc

_PAD = 1
_LANES = 16
_NW = 32
_K = 64
_KF = 128


@functools.lru_cache(maxsize=None)
def _build_sc_kernel(B, S, D):
    TOKW = (B * S) // _NW
    WPR = S // TOKW
    NCHUNK = TOKW // _K
    mesh = plsc.VectorSubcoreMesh(core_axis_name="c", subcore_axis_name="s")

    @functools.partial(
        pl.kernel,
        out_type=jax.ShapeDtypeStruct((B * S, D), jnp.float32),
        mesh=mesh,
        scratch_types=[
            pltpu.VMEM((S,), jnp.int32),
            pltpu.VMEM((TOKW,), jnp.int32),
            pltpu.VMEM((_K, D), jnp.float32),
            pltpu.SemaphoreType.DMA,
        ],
        compiler_params=pltpu.CompilerParams(
            needs_layout_passes=False, use_tc_tiling_on_sc=False
        ),
    )
    def sc_kernel(ids_hbm, table_hbm, out_hbm, ids_v, idx_v, rows_v, gsem):
        wid = lax.axis_index("s") * 2 + lax.axis_index("c")
        row = wid // WPR
        kk = wid % WPR

        pltpu.sync_copy(ids_hbm.at[pl.ds(row * S, S)], ids_v)

        one = jnp.full((_LANES,), 1, jnp.int32)
        zero = jnp.full((_LANES,), 0, jnp.int32)
        pad_vec = jnp.full((_LANES,), _PAD, jnp.int32)

        def _cnt(j, acc):
            v = ids_v[pl.ds(j * _LANES, _LANES)]
            mi = jnp.where(v != _PAD, one, zero)
            return acc + jnp.sum(mi)
        prefix = lax.fori_loop(0, kk * (TOKW // _LANES), _cnt, jnp.int32(0))

        chunk_off = kk * TOKW
        def _pos(j, run):
            v = ids_v[pl.ds(chunk_off + j * _LANES, _LANES)]
            m = v != _PAD
            mi = jnp.where(m, one, zero)
            c = jnp.cumsum(mi)
            idx_v[pl.ds(j * _LANES, _LANES)] = jnp.where(m, c + run, pad_vec)
            return run + jnp.sum(mi)
        lax.fori_loop(0, TOKW // _LANES, _pos, prefix + jnp.int32(1))

        out_base = wid * TOKW
        def _chunk(cix, carry):
            tok0 = cix * _KF
            def _pc(t, a):
                p = idx_v[pl.ds(tok0 + t * _LANES, _LANES)]
                return a + jnp.sum(jnp.where(p == _PAD, one, zero))
            npad = lax.fori_loop(0, _KF // _LANES, _pc, jnp.int32(0))
            first = idx_v[pl.ds(tok0, _LANES)][0]

            @pl.when(npad == 0)
            def _fast():
                pltpu.sync_copy(
                    table_hbm.at[pl.ds(first, _KF)],
                    out_hbm.at[pl.ds(out_base + tok0, _KF)],
                )

            @pl.when(npad != 0)
            def _slow():
                def _sub(s, c2):
                    idxs = idx_v.at[pl.ds(tok0 + s * _K, _K)]
                    pltpu.async_copy(table_hbm.at[idxs], rows_v, gsem).wait()
                    pltpu.sync_copy(
                        rows_v,
                        out_hbm.at[pl.ds(out_base + tok0 + s * _K, _K)],
                    )
                    return c2
                lax.fori_loop(0, _KF // _K, _sub, jnp.int32(0))
            return carry
        lax.fori_loop(0, TOKW // _KF, _chunk, jnp.int32(0))

    return sc_kernel


def kernel(input, weights):
    B, S = input.shape
    _, D = weights.shape
    out = _build_sc_kernel(B, S, D)(input.reshape(-1), weights)
    return out.reshape(B, S, D)


# speedup vs baseline: 35.1553x; 35.1553x over previous
import functools

import jax
import jax.numpy as jnp
from jax import lax
from jax.experimental import pallas as pl
from jax.experimental.pallas import tpu as pltpu
from jax.experimental.pallas import tpu_sc as plsc

_PAD = 1
_LANES = 16
_NW = 32
_K = 32


@functools.lru_cache(maxsize=None)
def _build_sc_kernel(B, S, D):
    TOKW = (B * S) // _NW
    WPR = S // TOKW
    NCHUNK = TOKW // _K
    mesh = plsc.VectorSubcoreMesh(core_axis_name="c", subcore_axis_name="s")

    @functools.partial(
        pl.kernel,
        out_type=jax.ShapeDtypeStruct((B * S, D), jnp.float32),
        mesh=mesh,
        scratch_types=[
            pltpu.VMEM((S,), jnp.int32),
            pltpu.VMEM((TOKW,), jnp.int32),
            pltpu.VMEM((_K, D), jnp.float32),
            pltpu.VMEM((_K, D), jnp.float32),
            pltpu.SemaphoreType.DMA,
        ],
        compiler_params=pltpu.CompilerParams(needs_layout_passes=False),
    )
    def sc_kernel(ids_hbm, table_hbm, out_hbm, ids_v, idx_v, rows0, rows1, gsem):
        wid = lax.axis_index("s") * 2 + lax.axis_index("c")
        row = wid // WPR
        kk = wid % WPR

        pltpu.sync_copy(ids_hbm.at[pl.ds(row * S, S)], ids_v)

        one = jnp.full((_LANES,), 1, jnp.int32)
        zero = jnp.full((_LANES,), 0, jnp.int32)
        pad_vec = jnp.full((_LANES,), _PAD, jnp.int32)

        def _cnt(j, acc):
            v = ids_v[pl.ds(j * _LANES, _LANES)]
            mi = jnp.where(v != _PAD, one, zero)
            return acc + jnp.sum(mi)
        prefix = lax.fori_loop(0, kk * (TOKW // _LANES), _cnt, jnp.int32(0))

        chunk_off = kk * TOKW
        def _pos(j, run):
            v = ids_v[pl.ds(chunk_off + j * _LANES, _LANES)]
            m = v != _PAD
            mi = jnp.where(m, one, zero)
            c = jnp.cumsum(mi)
            idx_v[pl.ds(j * _LANES, _LANES)] = jnp.where(m, c + run, pad_vec)
            return run + jnp.sum(mi)
        lax.fori_loop(0, TOKW // _LANES, _pos, prefix + jnp.int32(1))

        out_base = wid * TOKW
        pltpu.async_copy(table_hbm.at[idx_v.at[pl.ds(0, _K)]], rows0, gsem)

        def _pair(i, carry):
            for b, (buf, obuf) in enumerate(((rows0, rows1), (rows1, rows0))):
                cc = i * 2 + b
                pltpu.make_async_copy(
                    table_hbm.at[idx_v.at[pl.ds(0, _K)]], buf, gsem
                ).wait()

                @pl.when(cc + 1 < NCHUNK)
                def _start_next():
                    pltpu.async_copy(
                        table_hbm.at[idx_v.at[pl.ds((cc + 1) * _K, _K)]],
                        obuf,
                        gsem,
                    )

                pltpu.sync_copy(buf, out_hbm.at[pl.ds(out_base + cc * _K, _K)])
            return carry
        lax.fori_loop(0, NCHUNK // 2, _pair, jnp.int32(0))

    return sc_kernel


def kernel(input, weights):
    B, S = input.shape
    _, D = weights.shape
    out = _build_sc_kernel(B, S, D)(input.reshape(-1), weights)
    return out.reshape(B, S, D)


# speedup vs baseline: 35.3013x; 1.0042x over previous
import functools

import jax
import jax.numpy as jnp
from jax import lax
from jax.experimental import pallas as pl
from jax.experimental.pallas import tpu as pltpu
from jax.experimental.pallas import tpu_sc as plsc

_PAD = 1
_LANES = 16
_NW = 32
_K = 32


@functools.lru_cache(maxsize=None)
def _build_sc_kernel(B, S, D):
    TOKW = (B * S) // _NW
    WPR = S // TOKW
    NCHUNK = TOKW // _K
    mesh = plsc.VectorSubcoreMesh(core_axis_name="c", subcore_axis_name="s")

    @functools.partial(
        pl.kernel,
        out_type=jax.ShapeDtypeStruct((B * S, D), jnp.float32),
        mesh=mesh,
        scratch_types=[
            pltpu.VMEM((S,), jnp.int32),
            pltpu.VMEM((TOKW,), jnp.int32),
            pltpu.VMEM((_K, D), jnp.float32),
            pltpu.VMEM((_K, D), jnp.float32),
            pltpu.SemaphoreType.DMA,
            pltpu.SemaphoreType.DMA,
            pltpu.SemaphoreType.DMA,
        ],
        compiler_params=pltpu.CompilerParams(needs_layout_passes=False),
    )
    def sc_kernel(
        ids_hbm, table_hbm, out_hbm, ids_v, idx_v, rows0, rows1, gsem, osem0, osem1
    ):
        wid = lax.axis_index("s") * 2 + lax.axis_index("c")
        row = wid // WPR
        kk = wid % WPR

        pltpu.sync_copy(ids_hbm.at[pl.ds(row * S, S)], ids_v)

        one = jnp.full((_LANES,), 1, jnp.int32)
        zero = jnp.full((_LANES,), 0, jnp.int32)
        pad_vec = jnp.full((_LANES,), _PAD, jnp.int32)

        def _cnt(j, acc):
            v = ids_v[pl.ds(j * _LANES, _LANES)]
            return acc + jnp.where(v != _PAD, one, zero)
        accv = lax.fori_loop(0, kk * (TOKW // _LANES), _cnt, zero)
        prefix = jnp.sum(accv)

        chunk_off = kk * TOKW
        def _pos(j, run):
            v = ids_v[pl.ds(chunk_off + j * _LANES, _LANES)]
            m = v != _PAD
            c = jnp.cumsum(jnp.where(m, one, zero))
            idx_v[pl.ds(j * _LANES, _LANES)] = jnp.where(m, c + run, pad_vec)
            return run + c[_LANES - 1]
        lax.fori_loop(0, TOKW // _LANES, _pos, prefix + jnp.int32(1))

        out_base = wid * TOKW
        pltpu.async_copy(table_hbm.at[idx_v.at[pl.ds(0, _K)]], rows0, gsem)

        def _pair(i, carry):
            bufs = ((rows0, rows1, osem0, osem1), (rows1, rows0, osem1, osem0))
            for b, (buf, obuf, osem, oosem) in enumerate(bufs):
                cc = i * 2 + b
                pltpu.make_async_copy(
                    table_hbm.at[idx_v.at[pl.ds(0, _K)]], buf, gsem
                ).wait()
                pltpu.async_copy(
                    buf, out_hbm.at[pl.ds(out_base + cc * _K, _K)], osem
                )

                @pl.when(jnp.logical_and(cc + 1 < NCHUNK, cc >= 1))
                def _drain_prev():
                    pltpu.make_async_copy(
                        obuf, out_hbm.at[pl.ds(out_base, _K)], oosem
                    ).wait()

                @pl.when(cc + 1 < NCHUNK)
                def _start_next():
                    pltpu.async_copy(
                        table_hbm.at[idx_v.at[pl.ds((cc + 1) * _K, _K)]],
                        obuf,
                        gsem,
                    )
            return carry
        lax.fori_loop(0, NCHUNK // 2, _pair, jnp.int32(0))
        pltpu.make_async_copy(rows0, out_hbm.at[pl.ds(out_base, _K)], osem0).wait()
        pltpu.make_async_copy(rows1, out_hbm.at[pl.ds(out_base, _K)], osem1).wait()

    return sc_kernel


def kernel(input, weights):
    B, S = input.shape
    _, D = weights.shape
    out = _build_sc_kernel(B, S, D)(input.reshape(-1), weights)
    return out.reshape(B, S, D)


# speedup vs baseline: 41.1436x; 1.1655x over previous
import functools

import jax
import jax.numpy as jnp
from jax import lax
from jax.experimental import pallas as pl
from jax.experimental.pallas import tpu as pltpu
from jax.experimental.pallas import tpu_sc as plsc

_PAD = 1
_LANES = 16
_NW = 32
_K = 32


@functools.lru_cache(maxsize=None)
def _build_sc_kernel(B, S, D):
    SPAN = S // _NW
    NCHUNK = SPAN // _K
    mesh = plsc.VectorSubcoreMesh(core_axis_name="c", subcore_axis_name="s")

    @functools.partial(
        pl.kernel,
        out_type=jax.ShapeDtypeStruct((B * S, D), jnp.float32),
        mesh=mesh,
        scratch_types=[
            pltpu.VMEM((S,), jnp.int32),
            pltpu.VMEM((B * SPAN,), jnp.int32),
            pltpu.VMEM((_K, D), jnp.float32),
            pltpu.VMEM((_K, D), jnp.float32),
            pltpu.VMEM((_K, D), jnp.float32),
            pltpu.SemaphoreType.DMA,
            pltpu.SemaphoreType.DMA,
            pltpu.SemaphoreType.DMA,
        ],
        compiler_params=pltpu.CompilerParams(needs_layout_passes=False),
    )
    def sc_kernel(
        ids_hbm, table_hbm, out_hbm,
        ids_v, idx_v, bufa0, bufa1, bufb, gsem, osem0, osem1,
    ):
        wid = lax.axis_index("s") * 2 + lax.axis_index("c")
        tok0 = wid * SPAN

        one = jnp.full((_LANES,), 1, jnp.int32)
        zero = jnp.full((_LANES,), 0, jnp.int32)
        pad_vec = jnp.full((_LANES,), _PAD, jnp.int32)

        for r in range(B):
            pltpu.sync_copy(ids_hbm.at[pl.ds(r * S, S)], ids_v)

            def _cnt(j, acc):
                v = ids_v[pl.ds(j * _LANES, _LANES)]
                return acc + jnp.where(v != _PAD, one, zero)
            accv = lax.fori_loop(0, wid * (SPAN // _LANES), _cnt, zero)
            prefix = jnp.sum(accv)

            def _pos(j, run):
                v = ids_v[pl.ds(tok0 + j * _LANES, _LANES)]
                m = v != _PAD
                c = jnp.cumsum(jnp.where(m, one, zero))
                idx_v[pl.ds(r * SPAN + j * _LANES, _LANES)] = jnp.where(
                    m, c + run, pad_vec
                )
                return run + c[_LANES - 1]
            lax.fori_loop(0, SPAN // _LANES, _pos, prefix + jnp.int32(1))

        def _pair(i, carry):
            bufs = ((bufa0, osem0), (bufa1, osem1))
            for b, (buf, osem) in enumerate(bufs):
                cix = i * 2 + b
                cnt = carry[b]
                def _drain(_, c2):
                    pltpu.make_async_copy(
                        buf, out_hbm.at[pl.ds(tok0, _K)], osem
                    ).wait()
                    return c2
                lax.fori_loop(0, cnt, _drain, jnp.int32(0))

                coff = cix * _K
                pltpu.async_copy(
                    table_hbm.at[idx_v.at[pl.ds(coff, _K)]], buf, gsem
                ).wait()
                pltpu.async_copy(
                    buf, out_hbm.at[pl.ds(tok0 + coff, _K)], osem
                )
                ncopy = jnp.int32(1)
                for r in range(1, B):
                    roff = r * SPAN + coff
                    d0 = jnp.where(
                        idx_v[pl.ds(roff, _LANES)]
                        == idx_v[pl.ds(coff, _LANES)],
                        zero, one,
                    )
                    d1 = jnp.where(
                        idx_v[pl.ds(roff + _LANES, _LANES)]
                        == idx_v[pl.ds(coff + _LANES, _LANES)],
                        zero, one,
                    )
                    same = jnp.sum(d0 + d1) == 0
                    dst = out_hbm.at[pl.ds(r * S + tok0 + coff, _K)]

                    @pl.when(same)
                    def _reuse():
                        pltpu.async_copy(buf, dst, osem)

                    @pl.when(jnp.logical_not(same))
                    def _regather():
                        pltpu.async_copy(
                            table_hbm.at[idx_v.at[pl.ds(roff, _K)]],
                            bufb, gsem,
                        ).wait()
                        pltpu.sync_copy(bufb, dst)

                    ncopy = ncopy + jnp.where(same, jnp.int32(1), jnp.int32(0))
                if b == 0:
                    carry = (ncopy, carry[1])
                else:
                    carry = (carry[0], ncopy)
            return carry

        cnts = lax.fori_loop(
            0, NCHUNK // 2, _pair, (jnp.int32(0), jnp.int32(0))
        )
        def _drain0(_, c2):
            pltpu.make_async_copy(bufa0, out_hbm.at[pl.ds(tok0, _K)], osem0).wait()
            return c2
        lax.fori_loop(0, cnts[0], _drain0, jnp.int32(0))
        def _drain1(_, c2):
            pltpu.make_async_copy(bufa1, out_hbm.at[pl.ds(tok0, _K)], osem1).wait()
            return c2
        lax.fori_loop(0, cnts[1], _drain1, jnp.int32(0))

    return sc_kernel


def kernel(input, weights):
    B, S = input.shape
    _, D = weights.shape
    out = _build_sc_kernel(B, S, D)(input.reshape(-1), weights)
    return out.reshape(B, S, D)


# speedup vs baseline: 43.2367x; 1.0509x over previous
import functools

import jax
import jax.numpy as jnp
from jax import lax
from jax.experimental import pallas as pl
from jax.experimental.pallas import tpu as pltpu
from jax.experimental.pallas import tpu_sc as plsc

_PAD = 1
_LANES = 16
_NW = 32
_K = 32


@functools.lru_cache(maxsize=None)
def _build_sc_kernel(B, S, D):
    SPAN = S // _NW
    NCHUNK = SPAN // _K
    mesh = plsc.VectorSubcoreMesh(core_axis_name="c", subcore_axis_name="s")

    @functools.partial(
        pl.kernel,
        out_type=jax.ShapeDtypeStruct((B * S, D), jnp.float32),
        mesh=mesh,
        scratch_types=[
            pltpu.VMEM((S,), jnp.int32),
            pltpu.VMEM((S,), jnp.int32),
            pltpu.VMEM((B * SPAN,), jnp.int32),
            pltpu.VMEM((_K, D), jnp.float32),
            pltpu.VMEM((_K, D), jnp.float32),
            pltpu.VMEM((_K, D), jnp.float32),
            pltpu.SemaphoreType.DMA,
            pltpu.SemaphoreType.DMA,
            pltpu.SemaphoreType.DMA,
            pltpu.SemaphoreType.DMA,
        ],
        compiler_params=pltpu.CompilerParams(needs_layout_passes=False),
    )
    def sc_kernel(
        ids_hbm, table_hbm, out_hbm,
        ids0_v, ids1_v, idx_v, bufa0, bufa1, bufb, gsem, osem0, osem1, isem,
    ):
        wid = lax.axis_index("s") * 2 + lax.axis_index("c")
        tok0 = wid * SPAN

        one = jnp.full((_LANES,), 1, jnp.int32)
        zero = jnp.full((_LANES,), 0, jnp.int32)
        pad_vec = jnp.full((_LANES,), _PAD, jnp.int32)

        _UNROLL = 4
        pltpu.async_copy(ids_hbm.at[pl.ds(0, S)], ids0_v, isem)
        for r in range(B):
            ids_v = ids0_v if r % 2 == 0 else ids1_v
            nxt_v = ids1_v if r % 2 == 0 else ids0_v
            pltpu.make_async_copy(ids_hbm.at[pl.ds(0, S)], ids_v, isem).wait()
            if r + 1 < B:
                pltpu.async_copy(
                    ids_hbm.at[pl.ds((r + 1) * S, S)], nxt_v, isem
                )

            def _cnt(j, acc, ids_v=ids_v):
                for u in range(_UNROLL):
                    v = ids_v[pl.ds((j * _UNROLL + u) * _LANES, _LANES)]
                    acc = acc + jnp.where(v != _PAD, one, zero)
                return acc
            accv = lax.fori_loop(
                0, wid * (SPAN // (_UNROLL * _LANES)), _cnt, zero
            )
            prefix = jnp.sum(accv)

            def _pos(j, run, ids_v=ids_v, r=r):
                v = ids_v[pl.ds(tok0 + j * _LANES, _LANES)]
                m = v != _PAD
                c = jnp.cumsum(jnp.where(m, one, zero))
                idx_v[pl.ds(r * SPAN + j * _LANES, _LANES)] = jnp.where(
                    m, c + run, pad_vec
                )
                return run + c[_LANES - 1]
            lax.fori_loop(0, SPAN // _LANES, _pos, prefix + jnp.int32(1))

        def _pair(i, carry):
            bufs = ((bufa0, osem0), (bufa1, osem1))
            for b, (buf, osem) in enumerate(bufs):
                cix = i * 2 + b
                cnt = carry[b]
                def _drain(_, c2):
                    pltpu.make_async_copy(
                        buf, out_hbm.at[pl.ds(tok0, _K)], osem
                    ).wait()
                    return c2
                lax.fori_loop(0, cnt, _drain, jnp.int32(0))

                coff = cix * _K
                pltpu.async_copy(
                    table_hbm.at[idx_v.at[pl.ds(coff, _K)]], buf, gsem
                ).wait()
                pltpu.async_copy(
                    buf, out_hbm.at[pl.ds(tok0 + coff, _K)], osem
                )
                ncopy = jnp.int32(1)
                for r in range(1, B):
                    roff = r * SPAN + coff
                    d0 = jnp.where(
                        idx_v[pl.ds(roff, _LANES)]
                        == idx_v[pl.ds(coff, _LANES)],
                        zero, one,
                    )
                    d1 = jnp.where(
                        idx_v[pl.ds(roff + _LANES, _LANES)]
                        == idx_v[pl.ds(coff + _LANES, _LANES)],
                        zero, one,
                    )
                    same = jnp.sum(d0 + d1) == 0
                    dst = out_hbm.at[pl.ds(r * S + tok0 + coff, _K)]

                    @pl.when(same)
                    def _reuse():
                        pltpu.async_copy(buf, dst, osem)

                    @pl.when(jnp.logical_not(same))
                    def _regather():
                        pltpu.async_copy(
                            table_hbm.at[idx_v.at[pl.ds(roff, _K)]],
                            bufb, gsem,
                        ).wait()
                        pltpu.sync_copy(bufb, dst)

                    ncopy = ncopy + jnp.where(same, jnp.int32(1), jnp.int32(0))
                if b == 0:
                    carry = (ncopy, carry[1])
                else:
                    carry = (carry[0], ncopy)
            return carry

        cnts = lax.fori_loop(
            0, NCHUNK // 2, _pair, (jnp.int32(0), jnp.int32(0))
        )
        def _drain0(_, c2):
            pltpu.make_async_copy(bufa0, out_hbm.at[pl.ds(tok0, _K)], osem0).wait()
            return c2
        lax.fori_loop(0, cnts[0], _drain0, jnp.int32(0))
        def _drain1(_, c2):
            pltpu.make_async_copy(bufa1, out_hbm.at[pl.ds(tok0, _K)], osem1).wait()
            return c2
        lax.fori_loop(0, cnts[1], _drain1, jnp.int32(0))

    return sc_kernel


def kernel(input, weights):
    B, S = input.shape
    _, D = weights.shape
    out = _build_sc_kernel(B, S, D)(input.reshape(-1), weights)
    return out.reshape(B, S, D)
